# baseline (speedup 1.0000x reference)
import functools

import jax
import jax.numpy as jnp
from jax import lax
from jax.experimental import pallas as pl
from jax.experimental.pallas import tpu as pltpu
from jax.experimental.pallas import tpu_sc as plsc


def _project_body(cbfv_ref, w_ref, b_ref, out_ref):
    acc = lax.dot_general(
        cbfv_ref[...], w_ref[...],
        dimension_numbers=(((1,), (1,)), ((), ())),
        preferred_element_type=jnp.float32,
    )
    out_ref[...] = acc + b_ref[...][None, :]


def _project_table(cbfv, W, b):
    vocab = cbfv.shape[0]
    d_model = W.shape[0]
    return pl.pallas_call(
        _project_body,
        out_shape=jax.ShapeDtypeStruct((vocab, d_model), jnp.float32),
    )(cbfv, W, b)


_CB = 2
_NBUF = 4
_SEQ_PAD = 24


@functools.partial(jax.jit, static_argnums=(2, 3, 4))
def _sc_gather(table, idx_pad, batch, seq, d_model):
    try:
        info = plsc.get_sparse_core_info()
        nc, ns = info.num_cores, info.num_subcores
    except Exception:
        nc, ns = 2, 16
    nw = nc * ns
    assert batch % (nw * _CB) == 0
    b_per_w = batch // nw
    idx_per_w = b_per_w * _SEQ_PAD
    n_chunks = b_per_w // _CB
    assert (n_chunks - _NBUF) % _NBUF == 0 and n_chunks >= 2 * _NBUF

    mesh = plsc.VectorSubcoreMesh(core_axis_name="c", subcore_axis_name="s")

    @functools.partial(
        pl.kernel,
        mesh=mesh,
        out_type=jax.ShapeDtypeStruct((batch, seq, d_model), jnp.float32),
        scratch_types=[
            pltpu.VMEM((idx_per_w,), jnp.int32),
        ] + [pltpu.VMEM((_CB, seq, d_model), jnp.float32)] * _NBUF
          + [pltpu.SemaphoreType.DMA] * (2 * _NBUF),
    )
    def gather_kernel(table_hbm, idx_hbm, out_hbm, idx_v, *rest):
        bufs = rest[:_NBUF]
        gsems = rest[_NBUF:2 * _NBUF]
        osems = rest[2 * _NBUF:]
        wid = lax.axis_index("s") * nc + lax.axis_index("c")
        ibase = pl.multiple_of(wid * idx_per_w, idx_per_w)
        bbase = pl.multiple_of(wid * b_per_w, b_per_w)
        pltpu.sync_copy(idx_hbm.at[pl.ds(ibase, idx_per_w)], idx_v)

        def start_gather(j, b):
            for k in range(_CB):
                off = pl.multiple_of((j * _CB + k) * _SEQ_PAD, _SEQ_PAD)
                pltpu.async_copy(
                    table_hbm.at[idx_v.at[pl.ds(off, seq)]],
                    bufs[b].at[k], gsems[b])

        def start_out(j, b):
            pltpu.async_copy(bufs[b],
                             out_hbm.at[pl.ds(bbase + j * _CB, _CB)], osems[b])

        def wait_gather(b):
            for k in range(_CB):
                pltpu.make_async_copy(
                    table_hbm.at[idx_v.at[pl.ds(0, seq)]],
                    bufs[b].at[k], gsems[b]).wait()

        def wait_out(b):
            pltpu.make_async_copy(
                bufs[b], out_hbm.at[pl.ds(bbase, _CB)], osems[b]).wait()

        for b in range(_NBUF - 1):
            start_gather(b, b)
        wait_gather(0)
        start_out(0, 0)
        start_gather(_NBUF - 1, _NBUF - 1)

        def body(g, carry):
            for d in range(_NBUF):
                j = _NBUF * g + 1 + d
                b = (1 + d) % _NBUF
                wait_gather(b)
                start_out(j, b)
                wait_out((b + _NBUF - 1) % _NBUF)
                start_gather(j + _NBUF - 1, (b + _NBUF - 1) % _NBUF)
            return carry

        lax.fori_loop(0, (n_chunks - _NBUF) // _NBUF, body, 0)

        for j in range(n_chunks - _NBUF + 1, n_chunks):
            b = j % _NBUF
            wait_gather(b)
            start_out(j, b)
        for b in range(_NBUF):
            wait_out(b)

    return gather_kernel(table, idx_pad)


def kernel(src, cbfv, W, b):
    batch, seq = src.shape
    d_model = W.shape[0]
    table = _project_table(cbfv, W, b)
    idx = src.astype(jnp.int32)
    idx_pad = jnp.pad(idx, ((0, 0), (0, _SEQ_PAD - seq))).reshape(-1)
    return _sc_gather(table, idx_pad, batch, seq, d_model)
